# baseline (speedup 1.0000x reference)
import functools

import jax
import jax.numpy as jnp
from jax import lax
from jax.experimental import pallas as pl
from jax.experimental.pallas import tpu as pltpu
from jax.experimental.pallas import tpu_sc as plsc

_NC = 2
_NS = 16
_NW = _NC * _NS

_V = 1000000
_B = 16384 * 5
_N = 16384
_CHUNK = 128
_PER_W = _B // _NW
_NCHUNK = _PER_W // _CHUNK

def _make_gather():
  mesh = plsc.VectorSubcoreMesh(core_axis_name="c", subcore_axis_name="s")

  @functools.partial(
      pl.kernel,
      mesh=mesh,
      out_type=jax.ShapeDtypeStruct((_B, 128), jnp.float32),
      scratch_types=[
          pltpu.VMEM((_NCHUNK, _CHUNK), jnp.int32),
          pltpu.VMEM((_CHUNK, 128), jnp.float32),
          pltpu.VMEM((_CHUNK, 128), jnp.float32),
          pltpu.SemaphoreType.DMA,
          pltpu.SemaphoreType.DMA,
      ],
  )
  def gather_k(idx_hbm, table_hbm, out_hbm, idx_v, rows0, rows1, sem0, sem1):
    wid = lax.axis_index("s") * _NC + lax.axis_index("c")
    base = wid * _PER_W
    pltpu.sync_copy(idx_hbm.at[wid], idx_v)

    rows = (rows0, rows1)
    sems = (sem0, sem1)
    pltpu.async_copy(table_hbm.at[idx_v.at[0]], rows0, sem0)
    for j in range(_NCHUNK):
      if j + 1 < _NCHUNK:
        nxt = (j + 1) % 2
        pltpu.async_copy(table_hbm.at[idx_v.at[j + 1]], rows[nxt], sems[nxt])
      pltpu.make_async_copy(
          table_hbm.at[idx_v.at[j]], rows[j % 2], sems[j % 2]
      ).wait()
      pltpu.sync_copy(rows[j % 2], out_hbm.at[pl.ds(base + j * _CHUNK, _CHUNK)])

  return gather_k


_gather = _make_gather()

_MBLK = 2048


def _mm_body(g_ref, x_ref, w_ref, b_ref, o_ref):
  w = pl.program_id(1)
  v = x_ref[0, 0, :]
  odd = (v & 1).reshape(_MBLK, 1) == 1
  lane = lax.broadcasted_iota(jnp.int32, (_MBLK, 128), 1) >= 64
  g2 = jnp.where(lane == odd, g_ref[...], 0.0)
  acc = jnp.dot(g2, w_ref[...], preferred_element_type=jnp.float32)

  @pl.when(w == 0)
  def _():
    o_ref[...] = acc + b_ref[...]

  @pl.when(w != 0)
  def _():
    o_ref[...] += acc


def _matmul(g, xt3, w2, b_pad):
  nblk = _N // _MBLK
  return pl.pallas_call(
      _mm_body,
      grid=(nblk, 5),
      in_specs=[
          pl.BlockSpec((_MBLK, 128), lambda i, w: (w * (_N // _MBLK) + i, 0)),
          pl.BlockSpec((1, 1, _MBLK), lambda i, w: (w, 0, i)),
          pl.BlockSpec((128, 128), lambda i, w: (w, 0)),
          pl.BlockSpec((1, 128), lambda i, w: (0, 0)),
      ],
      out_specs=pl.BlockSpec((_MBLK, 128), lambda i, w: (i, 0)),
      out_shape=jax.ShapeDtypeStruct((_N, 128), jnp.float32),
  )(g, xt3, w2, b_pad)


@jax.jit
def kernel(x, emb_table, W, b):
  xt = x.T
  idx = (xt.reshape(-1) >> 1).reshape(_NW, _NCHUNK, _CHUNK)
  j_table = emb_table.reshape(_V // 2, 128)
  g = _gather(idx, j_table)
  wr = W.reshape(5, 64, 9)
  w2 = jnp.pad(
      jnp.concatenate([wr, wr], axis=1), ((0, 0), (0, 0), (0, 119))
  ).reshape(640, 128)
  b_pad = jnp.zeros((1, 128), jnp.float32).at[0, :9].set(b)
  out = _matmul(g, xt.reshape(5, 1, _N), w2, b_pad)
  return out[:, :9]


# speedup vs baseline: 1.0030x; 1.0030x over previous
import functools

import jax
import jax.numpy as jnp
from jax import lax
from jax.experimental import pallas as pl
from jax.experimental.pallas import tpu as pltpu
from jax.experimental.pallas import tpu_sc as plsc

_NC = 2
_NS = 16
_NW = _NC * _NS

_V = 1000000
_B = 16384 * 5
_N = 16384
_CHUNK = 128
_PER_W = _B // _NW
_NCHUNK = _PER_W // _CHUNK

def _make_gather():
  mesh = plsc.VectorSubcoreMesh(core_axis_name="c", subcore_axis_name="s")

  @functools.partial(
      pl.kernel,
      mesh=mesh,
      compiler_params=pltpu.CompilerParams(use_tc_tiling_on_sc=False),
      out_type=jax.ShapeDtypeStruct((_B, 128), jnp.float32),
      scratch_types=[
          pltpu.VMEM((_NCHUNK, _CHUNK), jnp.int32),
          pltpu.VMEM((_CHUNK, 128), jnp.float32),
          pltpu.VMEM((_CHUNK, 128), jnp.float32),
          pltpu.SemaphoreType.DMA,
          pltpu.SemaphoreType.DMA,
      ],
  )
  def gather_k(idx_hbm, table_hbm, out_hbm, idx_v, rows0, rows1, sem0, sem1):
    wid = lax.axis_index("s") * _NC + lax.axis_index("c")
    base = wid * _PER_W
    pltpu.sync_copy(idx_hbm.at[wid], idx_v)

    rows = (rows0, rows1)
    sems = (sem0, sem1)
    pltpu.async_copy(table_hbm.at[idx_v.at[0]], rows0, sem0)
    for j in range(_NCHUNK):
      if j + 1 < _NCHUNK:
        nxt = (j + 1) % 2
        pltpu.async_copy(table_hbm.at[idx_v.at[j + 1]], rows[nxt], sems[nxt])
      pltpu.make_async_copy(
          table_hbm.at[idx_v.at[j]], rows[j % 2], sems[j % 2]
      ).wait()
      pltpu.sync_copy(rows[j % 2], out_hbm.at[pl.ds(base + j * _CHUNK, _CHUNK)])

  return gather_k


_gather = _make_gather()

_MBLK = 2048


def _mm_body(g_ref, x_ref, w_ref, b_ref, o_ref):
  w = pl.program_id(1)
  v = x_ref[0, 0, :]
  odd = (v & 1).reshape(_MBLK, 1) == 1
  lane = lax.broadcasted_iota(jnp.int32, (_MBLK, 128), 1) >= 64
  g2 = jnp.where(lane == odd, g_ref[...], 0.0)
  acc = jnp.dot(g2, w_ref[...], preferred_element_type=jnp.float32)

  @pl.when(w == 0)
  def _():
    o_ref[...] = acc + b_ref[...]

  @pl.when(w != 0)
  def _():
    o_ref[...] += acc


def _matmul(g, xt3, w2, b_pad):
  nblk = _N // _MBLK
  return pl.pallas_call(
      _mm_body,
      grid=(nblk, 5),
      in_specs=[
          pl.BlockSpec((_MBLK, 128), lambda i, w: (w * (_N // _MBLK) + i, 0)),
          pl.BlockSpec((1, 1, _MBLK), lambda i, w: (w, 0, i)),
          pl.BlockSpec((128, 128), lambda i, w: (w, 0)),
          pl.BlockSpec((1, 128), lambda i, w: (0, 0)),
      ],
      out_specs=pl.BlockSpec((_MBLK, 128), lambda i, w: (i, 0)),
      out_shape=jax.ShapeDtypeStruct((_N, 128), jnp.float32),
  )(g, xt3, w2, b_pad)


@jax.jit
def kernel(x, emb_table, W, b):
  xt = x.T
  idx = (xt.reshape(-1) >> 1).reshape(_NW, _NCHUNK, _CHUNK)
  j_table = emb_table.reshape(_V // 2, 128)
  g = _gather(idx, j_table)
  wr = W.reshape(5, 64, 9)
  w2 = jnp.pad(
      jnp.concatenate([wr, wr], axis=1), ((0, 0), (0, 0), (0, 119))
  ).reshape(640, 128)
  b_pad = jnp.zeros((1, 128), jnp.float32).at[0, :9].set(b)
  out = _matmul(g, xt.reshape(5, 1, _N), w2, b_pad)
  return out[:, :9]


# speedup vs baseline: 1.3251x; 1.3212x over previous
import functools

import jax
import jax.numpy as jnp
from jax import lax
from jax.experimental import pallas as pl
from jax.experimental.pallas import tpu as pltpu
from jax.experimental.pallas import tpu_sc as plsc

_NC = 2
_NS = 16
_NW = _NC * _NS

_V = 1000000
_B = 16384 * 5
_N = 16384
_CHUNK = 128
_PER_W = _B // _NW
_NCHUNK = _PER_W // _CHUNK

_TBLK = 1024
_NTBLK = 489
_K = _TBLK * _NTBLK


def _transform_body(ta_ref, tb_ref, i_ref, o_ref):
  dims = (((0,), (0,)), ((), ()))
  ta = jax.lax.dot_general(
      ta_ref[...].astype(jnp.bfloat16), i_ref[...], dims,
      preferred_element_type=jnp.float32,
  )
  tb = jax.lax.dot_general(
      tb_ref[...].astype(jnp.bfloat16), i_ref[...], dims,
      preferred_element_type=jnp.float32,
  )
  o_ref[...] = jnp.concatenate([ta, tb], axis=1)


def _transform(tt, eye64):
  return pl.pallas_call(
      _transform_body,
      grid=(_NTBLK,),
      in_specs=[
          pl.BlockSpec((64, _TBLK), lambda i: (0, i)),
          pl.BlockSpec(
              (64, _TBLK),
              lambda i: (0, jnp.minimum(i + _NTBLK, _V // _TBLK)),
          ),
          pl.BlockSpec((64, 64), lambda i: (0, 0)),
      ],
      out_specs=pl.BlockSpec((_TBLK, 128), lambda i: (i, 0)),
      out_shape=jax.ShapeDtypeStruct((_K, 128), jnp.float32),
  )(tt, tt, eye64)


def _make_gather():
  mesh = plsc.VectorSubcoreMesh(core_axis_name="c", subcore_axis_name="s")

  @functools.partial(
      pl.kernel,
      mesh=mesh,
      out_type=jax.ShapeDtypeStruct((_B, 128), jnp.float32),
      scratch_types=[
          pltpu.VMEM((_NCHUNK, _CHUNK), jnp.int32),
          pltpu.VMEM((_CHUNK, 128), jnp.float32),
          pltpu.VMEM((_CHUNK, 128), jnp.float32),
          pltpu.SemaphoreType.DMA,
          pltpu.SemaphoreType.DMA,
      ],
  )
  def gather_k(idx_hbm, table_hbm, out_hbm, idx_v, rows0, rows1, sem0, sem1):
    wid = lax.axis_index("s") * _NC + lax.axis_index("c")
    base = wid * _PER_W
    pltpu.sync_copy(idx_hbm.at[wid], idx_v)

    rows = (rows0, rows1)
    sems = (sem0, sem1)
    pltpu.async_copy(table_hbm.at[idx_v.at[0]], rows0, sem0)
    for j in range(_NCHUNK):
      if j + 1 < _NCHUNK:
        nxt = (j + 1) % 2
        pltpu.async_copy(table_hbm.at[idx_v.at[j + 1]], rows[nxt], sems[nxt])
      pltpu.make_async_copy(
          table_hbm.at[idx_v.at[j]], rows[j % 2], sems[j % 2]
      ).wait()
      pltpu.sync_copy(rows[j % 2], out_hbm.at[pl.ds(base + j * _CHUNK, _CHUNK)])

  return gather_k


_gather = _make_gather()

_MBLK = 2048


def _mm_body(g_ref, x_ref, w_ref, b_ref, o_ref):
  w = pl.program_id(1)
  v = x_ref[0, 0, :]
  odd = jnp.where(v >= _K, 1, 0).reshape(_MBLK, 1) == 1
  lane = lax.broadcasted_iota(jnp.int32, (_MBLK, 128), 1) >= 64
  g2 = jnp.where(lane == odd, g_ref[...], 0.0)
  acc = jnp.dot(g2, w_ref[...], preferred_element_type=jnp.float32)

  @pl.when(w == 0)
  def _():
    o_ref[...] = acc + b_ref[...]

  @pl.when(w != 0)
  def _():
    o_ref[...] += acc


def _matmul(g, xt3, w2, b_pad):
  nblk = _N // _MBLK
  return pl.pallas_call(
      _mm_body,
      grid=(nblk, 5),
      in_specs=[
          pl.BlockSpec((_MBLK, 128), lambda i, w: (w * (_N // _MBLK) + i, 0)),
          pl.BlockSpec((1, 1, _MBLK), lambda i, w: (w, 0, i)),
          pl.BlockSpec((128, 128), lambda i, w: (w, 0)),
          pl.BlockSpec((1, 128), lambda i, w: (0, 0)),
      ],
      out_specs=pl.BlockSpec((_MBLK, 128), lambda i, w: (i, 0)),
      out_shape=jax.ShapeDtypeStruct((_N, 128), jnp.float32),
  )(g, xt3, w2, b_pad)


@jax.jit
def kernel(x, emb_table, W, b):
  xt = x.T
  xf = xt.reshape(-1)
  idx = jnp.where(xf >= _K, xf - _K, xf).reshape(_NW, _NCHUNK, _CHUNK)
  j_table = _transform(emb_table.T, jnp.eye(64, dtype=jnp.bfloat16))
  g = _gather(idx, j_table)
  wr = W.reshape(5, 64, 9)
  w2 = jnp.pad(
      jnp.concatenate([wr, wr], axis=1), ((0, 0), (0, 0), (0, 119))
  ).reshape(640, 128)
  b_pad = jnp.zeros((1, 128), jnp.float32).at[0, :9].set(b)
  out = _matmul(g, xt.reshape(5, 1, _N), w2, b_pad)
  return out[:, :9]


# speedup vs baseline: 2.2470x; 1.6957x over previous
import functools

import jax
import jax.numpy as jnp
from jax import lax
from jax.experimental import pallas as pl
from jax.experimental.pallas import tpu as pltpu
from jax.experimental.pallas import tpu_sc as plsc

_NC = 2
_NS = 16
_NW = _NC * _NS

_V = 1000000
_B = 16384 * 5
_N = 16384
_CHUNK = 128
_PER_W = _B // _NW
_NCHUNK = _PER_W // _CHUNK

_TBLK = 4096
_NTBLK = 123
_K = _TBLK * _NTBLK


def _transform_body(ta_ref, tb_ref, i_ref, o_ref):
  dims = (((0,), (0,)), ((), ()))
  ta = jax.lax.dot_general(
      ta_ref[...].astype(jnp.bfloat16), i_ref[...], dims,
      preferred_element_type=jnp.float32,
  )
  tb = jax.lax.dot_general(
      tb_ref[...].astype(jnp.bfloat16), i_ref[...], dims,
      preferred_element_type=jnp.float32,
  )
  o_ref[...] = jnp.concatenate([ta, tb], axis=1)


def _transform(tt, eye64):
  return pl.pallas_call(
      _transform_body,
      grid=(_NTBLK,),
      in_specs=[
          pl.BlockSpec((64, _TBLK), lambda i: (0, i)),
          pl.BlockSpec(
              (64, _TBLK),
              lambda i: (0, jnp.minimum(i + _NTBLK, _V // _TBLK)),
          ),
          pl.BlockSpec((64, 64), lambda i: (0, 0)),
      ],
      out_specs=pl.BlockSpec((_TBLK, 128), lambda i: (i, 0)),
      out_shape=jax.ShapeDtypeStruct((_K, 128), jnp.float32),
  )(tt, tt, eye64)


def _make_gather():
  mesh = plsc.VectorSubcoreMesh(core_axis_name="c", subcore_axis_name="s")

  @functools.partial(
      pl.kernel,
      mesh=mesh,
      out_type=jax.ShapeDtypeStruct((_B, 128), jnp.float32),
      scratch_types=[
          pltpu.VMEM((_NCHUNK, _CHUNK), jnp.int32),
          pltpu.VMEM((_CHUNK, 128), jnp.float32),
          pltpu.VMEM((_CHUNK, 128), jnp.float32),
          pltpu.SemaphoreType.DMA,
          pltpu.SemaphoreType.DMA,
      ],
  )
  def gather_k(idx_hbm, table_hbm, out_hbm, idx_v, rows0, rows1, sem0, sem1):
    wid = lax.axis_index("s") * _NC + lax.axis_index("c")
    base = wid * _PER_W
    pltpu.sync_copy(idx_hbm.at[wid], idx_v)

    rows = (rows0, rows1)
    sems = (sem0, sem1)
    pltpu.async_copy(table_hbm.at[idx_v.at[0]], rows0, sem0)
    for j in range(_NCHUNK):
      if j + 1 < _NCHUNK:
        nxt = (j + 1) % 2
        pltpu.async_copy(table_hbm.at[idx_v.at[j + 1]], rows[nxt], sems[nxt])
      pltpu.make_async_copy(
          table_hbm.at[idx_v.at[j]], rows[j % 2], sems[j % 2]
      ).wait()
      pltpu.sync_copy(rows[j % 2], out_hbm.at[pl.ds(base + j * _CHUNK, _CHUNK)])

  return gather_k


_gather = _make_gather()

_MBLK = 8192


def _mm_body(g_ref, x_ref, w_ref, b_ref, o_ref):
  w = pl.program_id(1)
  v = x_ref[0, 0, :]
  odd = jnp.where(v >= _K, 1, 0).reshape(_MBLK, 1) == 1
  lane = lax.broadcasted_iota(jnp.int32, (_MBLK, 128), 1) >= 64
  g2 = jnp.where(lane == odd, g_ref[...], 0.0)
  acc = jnp.dot(g2, w_ref[...], preferred_element_type=jnp.float32)

  @pl.when(w == 0)
  def _():
    o_ref[...] = acc + b_ref[...]

  @pl.when(w != 0)
  def _():
    o_ref[...] += acc


def _matmul(g, xt3, w2, b_pad):
  nblk = _N // _MBLK
  return pl.pallas_call(
      _mm_body,
      grid=(nblk, 5),
      in_specs=[
          pl.BlockSpec((_MBLK, 128), lambda i, w: (w * (_N // _MBLK) + i, 0)),
          pl.BlockSpec((1, 1, _MBLK), lambda i, w: (w, 0, i)),
          pl.BlockSpec((128, 128), lambda i, w: (w, 0)),
          pl.BlockSpec((1, 128), lambda i, w: (0, 0)),
      ],
      out_specs=pl.BlockSpec((_MBLK, 128), lambda i, w: (i, 0)),
      out_shape=jax.ShapeDtypeStruct((_N, 128), jnp.float32),
  )(g, xt3, w2, b_pad)


@jax.jit
def kernel(x, emb_table, W, b):
  xt = x.T
  xf = xt.reshape(-1)
  idx = jnp.where(xf >= _K, xf - _K, xf).reshape(_NW, _NCHUNK, _CHUNK)
  j_table = _transform(emb_table.T, jnp.eye(64, dtype=jnp.bfloat16))
  g = _gather(idx, j_table)
  wr = W.reshape(5, 64, 9)
  w2 = jnp.pad(
      jnp.concatenate([wr, wr], axis=1), ((0, 0), (0, 0), (0, 119))
  ).reshape(640, 128)
  b_pad = jnp.zeros((1, 128), jnp.float32).at[0, :9].set(b)
  out = _matmul(g, xt.reshape(5, 1, _N), w2, b_pad)
  return out[:, :9]


# speedup vs baseline: 2.5397x; 1.1303x over previous
import functools

import jax
import jax.numpy as jnp
from jax import lax
from jax.experimental import pallas as pl
from jax.experimental.pallas import tpu as pltpu
from jax.experimental.pallas import tpu_sc as plsc

_NC = 2
_NS = 16
_NW = _NC * _NS

_V = 1000000
_B = 16384 * 5
_N = 16384
_CHUNK = 128
_PER_W = _B // _NW
_NCHUNK = _PER_W // _CHUNK

_TBLK = 8192
_NTBLK = 62
_K = _TBLK * _NTBLK


def _transform_body(ta_ref, tb_ref, i_ref, o_ref):
  dims = (((0,), (0,)), ((), ()))
  ta = jax.lax.dot_general(
      ta_ref[...].astype(jnp.bfloat16), i_ref[...], dims,
      preferred_element_type=jnp.float32,
  )
  tb = jax.lax.dot_general(
      tb_ref[...].astype(jnp.bfloat16), i_ref[...], dims,
      preferred_element_type=jnp.float32,
  )
  o_ref[...] = jnp.concatenate([ta, tb], axis=1)


def _transform(tt, eye64):
  return pl.pallas_call(
      _transform_body,
      grid=(_NTBLK,),
      in_specs=[
          pl.BlockSpec((64, _TBLK), lambda i: (0, i)),
          pl.BlockSpec(
              (64, _TBLK),
              lambda i: (0, jnp.minimum(i + _NTBLK, _V // _TBLK)),
          ),
          pl.BlockSpec((64, 64), lambda i: (0, 0)),
      ],
      out_specs=pl.BlockSpec((_TBLK, 128), lambda i: (i, 0)),
      out_shape=jax.ShapeDtypeStruct((_K, 128), jnp.float32),
  )(tt, tt, eye64)


def _make_gather():
  mesh = plsc.VectorSubcoreMesh(core_axis_name="c", subcore_axis_name="s")

  @functools.partial(
      pl.kernel,
      mesh=mesh,
      out_type=jax.ShapeDtypeStruct((_B, 128), jnp.float32),
      scratch_types=[
          pltpu.VMEM((_NCHUNK, _CHUNK), jnp.int32),
          pltpu.VMEM((_CHUNK, 128), jnp.float32),
          pltpu.VMEM((_CHUNK, 128), jnp.float32),
          pltpu.SemaphoreType.DMA,
          pltpu.SemaphoreType.DMA,
      ],
  )
  def gather_k(idx_hbm, table_hbm, out_hbm, idx_v, rows0, rows1, sem0, sem1):
    wid = lax.axis_index("s") * _NC + lax.axis_index("c")
    base = wid * _PER_W
    pltpu.sync_copy(idx_hbm.at[wid], idx_v)

    rows = (rows0, rows1)
    sems = (sem0, sem1)
    pltpu.async_copy(table_hbm.at[idx_v.at[0]], rows0, sem0)
    for j in range(_NCHUNK):
      if j + 1 < _NCHUNK:
        nxt = (j + 1) % 2
        pltpu.async_copy(table_hbm.at[idx_v.at[j + 1]], rows[nxt], sems[nxt])
      pltpu.make_async_copy(
          table_hbm.at[idx_v.at[j]], rows[j % 2], sems[j % 2]
      ).wait()
      pltpu.sync_copy(rows[j % 2], out_hbm.at[pl.ds(base + j * _CHUNK, _CHUNK)])

  return gather_k


_gather = _make_gather()

_MBLK = 8192


def _mm_body(g_ref, x_ref, w_ref, b_ref, o_ref):
  w = pl.program_id(1)
  v = x_ref[0, 0, :]
  odd = jnp.where(v >= _K, 1, 0).reshape(_MBLK, 1) == 1
  lane = lax.broadcasted_iota(jnp.int32, (_MBLK, 128), 1) >= 64
  g2 = jnp.where(lane == odd, g_ref[...], 0.0)
  acc = jnp.dot(g2, w_ref[...], preferred_element_type=jnp.float32)

  @pl.when(w == 0)
  def _():
    o_ref[...] = acc + b_ref[...]

  @pl.when(w != 0)
  def _():
    o_ref[...] += acc


def _matmul(g, xt3, w2, b_pad):
  nblk = _N // _MBLK
  return pl.pallas_call(
      _mm_body,
      grid=(nblk, 5),
      in_specs=[
          pl.BlockSpec((_MBLK, 128), lambda i, w: (w * (_N // _MBLK) + i, 0)),
          pl.BlockSpec((1, 1, _MBLK), lambda i, w: (w, 0, i)),
          pl.BlockSpec((128, 128), lambda i, w: (w, 0)),
          pl.BlockSpec((1, 128), lambda i, w: (0, 0)),
      ],
      out_specs=pl.BlockSpec((_MBLK, 128), lambda i, w: (i, 0)),
      out_shape=jax.ShapeDtypeStruct((_N, 128), jnp.float32),
  )(g, xt3, w2, b_pad)


@jax.jit
def kernel(x, emb_table, W, b):
  xt = x.T
  xf = xt.reshape(-1)
  idx = jnp.where(xf >= _K, xf - _K, xf).reshape(_NW, _NCHUNK, _CHUNK)
  j_table = _transform(emb_table.T, jnp.eye(64, dtype=jnp.bfloat16))
  g = _gather(idx, j_table)
  wr = W.reshape(5, 64, 9)
  w2 = jnp.pad(
      jnp.concatenate([wr, wr], axis=1), ((0, 0), (0, 0), (0, 119))
  ).reshape(640, 128)
  b_pad = jnp.zeros((1, 128), jnp.float32).at[0, :9].set(b)
  out = _matmul(g, xt.reshape(5, 1, _N), w2, b_pad)
  return out[:, :9]


# speedup vs baseline: 2.7214x; 1.0715x over previous
import functools

import jax
import jax.numpy as jnp
from jax import lax
from jax.experimental import pallas as pl
from jax.experimental.pallas import tpu as pltpu
from jax.experimental.pallas import tpu_sc as plsc

_NC = 2
_NS = 16
_NW = _NC * _NS

_V = 1000000
_B = 16384 * 5
_N = 16384
_CHUNK = 128
_PER_W = _B // _NW
_NCHUNK = _PER_W // _CHUNK

_TBLK = 16384
_NTBLK = 31
_K = _TBLK * _NTBLK


def _transform_body(ta_ref, tb_ref, i_ref, o_ref):
  dims = (((0,), (0,)), ((), ()))
  ta = jax.lax.dot_general(
      ta_ref[...].astype(jnp.bfloat16), i_ref[...], dims,
      preferred_element_type=jnp.float32,
  )
  tb = jax.lax.dot_general(
      tb_ref[...].astype(jnp.bfloat16), i_ref[...], dims,
      preferred_element_type=jnp.float32,
  )
  o_ref[...] = jnp.concatenate([ta, tb], axis=1)


def _transform(tt, eye64):
  return pl.pallas_call(
      _transform_body,
      grid=(_NTBLK,),
      in_specs=[
          pl.BlockSpec((64, _TBLK), lambda i: (0, i)),
          pl.BlockSpec(
              (64, _TBLK),
              lambda i: (0, jnp.minimum(i + _NTBLK, _V // _TBLK)),
          ),
          pl.BlockSpec((64, 64), lambda i: (0, 0)),
      ],
      out_specs=pl.BlockSpec((_TBLK, 128), lambda i: (i, 0)),
      out_shape=jax.ShapeDtypeStruct((_K, 128), jnp.float32),
  )(tt, tt, eye64)


def _make_gather():
  mesh = plsc.VectorSubcoreMesh(core_axis_name="c", subcore_axis_name="s")

  @functools.partial(
      pl.kernel,
      mesh=mesh,
      out_type=jax.ShapeDtypeStruct((_B, 128), jnp.float32),
      scratch_types=[
          pltpu.VMEM((_NCHUNK, _CHUNK), jnp.int32),
          pltpu.VMEM((_CHUNK, 128), jnp.float32),
          pltpu.VMEM((_CHUNK, 128), jnp.float32),
          pltpu.SemaphoreType.DMA,
          pltpu.SemaphoreType.DMA,
      ],
  )
  def gather_k(idx_hbm, table_hbm, out_hbm, idx_v, rows0, rows1, sem0, sem1):
    wid = lax.axis_index("s") * _NC + lax.axis_index("c")
    base = wid * _PER_W
    pltpu.sync_copy(idx_hbm.at[wid], idx_v)

    rows = (rows0, rows1)
    sems = (sem0, sem1)
    pltpu.async_copy(table_hbm.at[idx_v.at[0]], rows0, sem0)
    for j in range(_NCHUNK):
      if j + 1 < _NCHUNK:
        nxt = (j + 1) % 2
        pltpu.async_copy(table_hbm.at[idx_v.at[j + 1]], rows[nxt], sems[nxt])
      pltpu.make_async_copy(
          table_hbm.at[idx_v.at[j]], rows[j % 2], sems[j % 2]
      ).wait()
      pltpu.sync_copy(rows[j % 2], out_hbm.at[pl.ds(base + j * _CHUNK, _CHUNK)])

  return gather_k


_gather = _make_gather()

_MBLK = 8192


def _mm_body(g_ref, x_ref, w_ref, b_ref, o_ref):
  w = pl.program_id(1)
  v = x_ref[0, 0, :]
  odd = jnp.where(v >= _K, 1, 0).reshape(_MBLK, 1) == 1
  lane = lax.broadcasted_iota(jnp.int32, (_MBLK, 128), 1) >= 64
  g2 = jnp.where(lane == odd, g_ref[...], 0.0)
  acc = jnp.dot(g2, w_ref[...], preferred_element_type=jnp.float32)

  @pl.when(w == 0)
  def _():
    o_ref[...] = acc + b_ref[...]

  @pl.when(w != 0)
  def _():
    o_ref[...] += acc


def _matmul(g, xt3, w2, b_pad):
  nblk = _N // _MBLK
  return pl.pallas_call(
      _mm_body,
      grid=(nblk, 5),
      in_specs=[
          pl.BlockSpec((_MBLK, 128), lambda i, w: (w * (_N // _MBLK) + i, 0)),
          pl.BlockSpec((1, 1, _MBLK), lambda i, w: (w, 0, i)),
          pl.BlockSpec((128, 128), lambda i, w: (w, 0)),
          pl.BlockSpec((1, 128), lambda i, w: (0, 0)),
      ],
      out_specs=pl.BlockSpec((_MBLK, 128), lambda i, w: (i, 0)),
      out_shape=jax.ShapeDtypeStruct((_N, 128), jnp.float32),
  )(g, xt3, w2, b_pad)


@jax.jit
def kernel(x, emb_table, W, b):
  xt = x.T
  xf = xt.reshape(-1)
  idx = jnp.where(xf >= _K, xf - _K, xf).reshape(_NW, _NCHUNK, _CHUNK)
  j_table = _transform(emb_table.T, jnp.eye(64, dtype=jnp.bfloat16))
  g = _gather(idx, j_table)
  wr = W.reshape(5, 64, 9)
  w2 = jnp.pad(
      jnp.concatenate([wr, wr], axis=1), ((0, 0), (0, 0), (0, 119))
  ).reshape(640, 128)
  b_pad = jnp.zeros((1, 128), jnp.float32).at[0, :9].set(b)
  out = _matmul(g, xt.reshape(5, 1, _N), w2, b_pad)
  return out[:, :9]
